# baseline (speedup 1.0000x reference)
import functools

import jax
import jax.numpy as jnp
from jax import lax
from jax.experimental import pallas as pl
from jax.experimental.pallas import tpu as pltpu
from jax.experimental.pallas import tpu_sc as plsc

V = 10000
D = 128
E = 600000
EP = 300000
NP = 10240
NC = 2
NS = 16
NW = NC * NS
CH = 128
PCH0, PCH1 = 72, 80
ACH0, ACH1 = 72, 80
PCH_MAX = max(PCH0, PCH1)
NCH_MAX = max(ACH0, ACH1)
ROWS = NS * (PCH0 + PCH1)
EPAD = ROWS * CH
ROWS_ALLOC = ROWS
RPT = NP // NS
ITERS = 2
_f32 = jnp.float32
_i32 = jnp.int32

_mesh = plsc.VectorSubcoreMesh(core_axis_name="c", subcore_axis_name="s")


@functools.partial(
    pl.kernel,
    out_type=(
        jax.ShapeDtypeStruct((ROWS_ALLOC, CH), _i32),
        jax.ShapeDtypeStruct((ROWS_ALLOC, CH), _i32),
        jax.ShapeDtypeStruct((ROWS_ALLOC, CH), _i32),
        jax.ShapeDtypeStruct((ROWS_ALLOC, CH), _i32),
        jax.ShapeDtypeStruct((NC, 4, NP), _f32),
    ),
    mesh=_mesh,
    scratch_types=(
        pltpu.VMEM((PCH_MAX, CH), _i32),
        pltpu.VMEM((PCH_MAX, CH), _i32),
        pltpu.VMEM((PCH_MAX, CH), _i32),
        pltpu.VMEM((CH,), _f32),
        pltpu.VMEM((RPT,), _f32),
        pltpu.VMEM_SHARED((NP,), _f32),
        pltpu.VMEM_SHARED((NP,), _f32),
        pltpu.VMEM_SHARED((NP,), _f32),
        pltpu.VMEM_SHARED((NP,), _f32),
        tuple(pltpu.SemaphoreType.DMA for _ in range(4)),
        pltpu.SemaphoreType.DMA,
    ),
)
def _sc_prep(v_ei, c_ei, p_idx, n_idx,
             pv_out, pc_out, nv_out, nc_out, cnt_out,
             ebuf, vall, call_, ones, zrows, h_pv, h_pc, h_nv, h_nc, gs, sh):
    cid = lax.axis_index("c")
    sid = lax.axis_index("s")
    pch = PCH0 + cid * (PCH1 - PCH0)
    rb = pl.multiple_of(cid * NS * PCH0 + sid * pch, 8)
    for q in range(CH // 16):
        ones[pl.ds(q * 16, 16)] = jnp.ones((16,), _f32)

    @pl.loop(0, RPT // 16)
    def _(q):
        zrows[pl.ds(q * 16, 16)] = jnp.zeros((16,), _f32)

    for h in (h_pv, h_pc, h_nv, h_nc):
        pltpu.sync_copy(zrows, h.at[pl.ds(sid * RPT, RPT)])
    plsc.subcore_barrier()

    def run(eidx, v_out, c_out, hv, hc):
        pltpu.sync_copy(eidx.at[pl.ds(rb, PCH_MAX)], ebuf)

        def fire(j, b):
            pltpu.async_copy(v_ei.at[ebuf.at[j]], vall.at[j], gs[b])
            pltpu.async_copy(c_ei.at[ebuf.at[j]], call_.at[j], gs[b])

        def drain(b):
            pltpu.make_async_copy(v_ei.at[ebuf.at[0]], vall.at[0],
                                  gs[b]).wait()
            pltpu.make_async_copy(c_ei.at[ebuf.at[0]], call_.at[0],
                                  gs[b]).wait()

        for b in range(3):
            fire(b, b)

        @pl.loop(0, pch // 4)
        def _(t):
            for b in range(4):
                j = 4 * t + b

                @pl.when(j + 3 < pch)
                def _():
                    fire(j + 3, (b + 3) % 4)

                drain(b)
                pltpu.async_copy(ones, hv.at[vall.at[j]], sh, add=True)
                pltpu.async_copy(ones, hc.at[call_.at[j]], sh, add=True)

        @pl.loop(0, 2 * pch)
        def _(j):
            pltpu.make_async_copy(ones, hv.at[vall.at[0]], sh).wait()

        pltpu.sync_copy(vall.at[pl.ds(0, PCH0)], v_out.at[pl.ds(rb, PCH0)])
        pltpu.sync_copy(call_.at[pl.ds(0, PCH0)], c_out.at[pl.ds(rb, PCH0)])

        @pl.when(pch > PCH0)
        def _():
            rb2 = pl.multiple_of(rb + PCH0, 8)
            pltpu.sync_copy(vall.at[pl.ds(PCH0, PCH_MAX - PCH0)],
                            v_out.at[pl.ds(rb2, PCH_MAX - PCH0)])
            pltpu.sync_copy(call_.at[pl.ds(PCH0, PCH_MAX - PCH0)],
                            c_out.at[pl.ds(rb2, PCH_MAX - PCH0)])

    run(p_idx, pv_out, pc_out, h_pv, h_pc)
    run(n_idx, nv_out, nc_out, h_nv, h_nc)
    plsc.subcore_barrier()
    for a, h in enumerate((h_pv, h_pc, h_nv, h_nc)):
        pltpu.sync_copy(h.at[pl.ds(sid * RPT, RPT)],
                        cnt_out.at[cid, a, pl.ds(sid * RPT, RPT)])


@functools.partial(
    pl.kernel,
    out_type=tuple(jax.ShapeDtypeStruct((NC, NP, D), _f32) for _ in range(4)),
    mesh=_mesh,
    scratch_types=(
        tuple(pltpu.VMEM((CH,), _i32) for _ in range(2)),
        tuple(pltpu.VMEM((CH,), _i32) for _ in range(2)),
        tuple(pltpu.VMEM((CH, D), _f32) for _ in range(2)),
        pltpu.VMEM((16, D), _f32),
        pltpu.VMEM_SHARED((NP, D), _f32),
        tuple(pltpu.SemaphoreType.DMA for _ in range(2)),
        tuple(pltpu.SemaphoreType.DMA for _ in range(2)),
    ),
)
def _sc_aggr(m_pv, m_nv, m_pc, m_nc, pv, pc, nv, nc,
             o_pvc, o_nvc, o_pcv, o_ncv,
             sbuf, dbuf, rows, zblk, acc, si, gs):
    cid = lax.axis_index("c")
    sid = lax.axis_index("s")
    nch = ACH0 + cid * (ACH1 - ACH0)
    eb = pl.multiple_of((cid * NS * ACH0 + sid * nch) * CH, 1024)

    @pl.loop(0, 16)
    def _(r):
        for q in range(D // 16):
            zblk[r, pl.ds(q * 16, 16)] = jnp.zeros((16,), _f32)

    jobs = ((m_pv, pv, pc, o_pvc),
            (m_nv, nv, nc, o_nvc),
            (m_pc, pc, pv, o_pcv),
            (m_nc, nc, nv, o_ncv))
    for tab, src, dst, out in jobs:
        @pl.loop(0, RPT // 16)
        def _(b):
            pltpu.sync_copy(zblk, acc.at[pl.ds(sid * RPT + b * 16, 16)])
        plsc.subcore_barrier()

        def startidx(j, b):
            pltpu.async_copy(src.at[pl.ds(eb + j * CH, CH)], sbuf[b], si[b])
            pltpu.async_copy(dst.at[pl.ds(eb + j * CH, CH)], dbuf[b], si[b])

        def startgather(b):
            pltpu.make_async_copy(src.at[pl.ds(0, CH)], sbuf[b],
                                  si[b]).wait()
            pltpu.make_async_copy(dst.at[pl.ds(0, CH)], dbuf[b],
                                  si[b]).wait()
            pltpu.async_copy(tab.at[sbuf[b]], rows[b], gs[b])

        def finish(b):
            pltpu.make_async_copy(tab.at[sbuf[b]], rows[b], gs[b]).wait()
            pltpu.sync_copy(rows[b], acc.at[dbuf[b]], add=True)

        startidx(0, 0)
        startidx(1, 1)
        startgather(0)

        @pl.loop(0, nch // 2)
        def _(t):
            for b in range(2):
                j = 2 * t + b
                o = (b + 1) % 2

                @pl.when(j + 1 < nch)
                def _():
                    startgather(o)

                finish(b)

                @pl.when(j + 2 < nch)
                def _():
                    startidx(j + 2, b)

        plsc.subcore_barrier()
        pltpu.sync_copy(acc.at[pl.ds(sid * RPT, RPT)],
                        out.at[cid, pl.ds(sid * RPT, RPT)])
        plsc.subcore_barrier()


_BLK = 128
_G = NP // _BLK


def _dot(a, b):
    return jnp.dot(a, b, preferred_element_type=_f32,
                   precision=lax.Precision.HIGHEST)


def _msg_body(v_ref, c_ref, cnt_ref,
              pw1, pb1, pw2, pb2, nw1, nb1, nw2, nb2,
              qw1, qb1, qw2, qb2, rw1, rb1, rw2, rb2,
              o_pv, o_nv, o_pc, o_nc):
    cnt = cnt_ref[...]

    def scale(a):
        return lax.rsqrt(jnp.maximum(cnt[a] + cnt[4 + a], 1.0))

    def mlp(x, w1, b1, w2, b2):
        h = jnp.maximum(_dot(x, w1[...]) + b1[...], 0.0)
        return _dot(h, w2[...]) + b2[...]

    xv = v_ref[...]
    xc = c_ref[...]
    o_pv[...] = mlp(xv, pw1, pb1, pw2, pb2) * scale(0)[:, None]
    o_nv[...] = mlp(xv, nw1, nb1, nw2, nb2) * scale(2)[:, None]
    o_pc[...] = mlp(xc, qw1, qb1, qw2, qb2) * scale(1)[:, None]
    o_nc[...] = mlp(xc, rw1, rb1, rw2, rb2) * scale(3)[:, None]


def _tc_msg(vp, cp, cnt8, *ws):
    row = pl.BlockSpec((_BLK, D), lambda i: (i, 0))
    cnt = pl.BlockSpec((8, _BLK), lambda i: (0, i))
    w = pl.BlockSpec((D, D), lambda i: (0, 0))
    b = pl.BlockSpec((1, D), lambda i: (0, 0))
    return pl.pallas_call(
        _msg_body,
        grid=(_G,),
        in_specs=[row, row, cnt] + [w, b, w, b] * 4,
        out_specs=[row] * 4,
        out_shape=[jax.ShapeDtypeStruct((NP, D), _f32)] * 4,
    )(vp, cp, cnt8, *ws)


def _upd_body(c_ref, v_ref, a0, a1, a2, a3, cnt_ref,
              wc, bc, wv, bv, oc, ov):
    cnt = cnt_ref[...]

    def scale(a):
        return lax.rsqrt(jnp.maximum(cnt[a] + cnt[4 + a], 1.0))

    def agg(aref, a):
        x = aref[...]
        return (x[0] + x[1]) * scale(a)[:, None]

    wcm = wc[...]
    wvm = wv[...]
    oc[...] = (_dot(c_ref[...], wcm[0:D]) + _dot(agg(a0, 1), wcm[D:2 * D])
               + _dot(agg(a1, 3), wcm[2 * D:3 * D]) + bc[...])
    ov[...] = (_dot(v_ref[...], wvm[0:D]) + _dot(agg(a2, 0), wvm[D:2 * D])
               + _dot(agg(a3, 2), wvm[2 * D:3 * D]) + bv[...])


def _tc_upd(cp, vp, a_pvc, a_nvc, a_pcv, a_ncv, cnt8, wc, bc, wv, bv):
    row = pl.BlockSpec((_BLK, D), lambda i: (i, 0))
    aspec = pl.BlockSpec((NC, _BLK, D), lambda i: (0, i, 0))
    cnt = pl.BlockSpec((8, _BLK), lambda i: (0, i))
    w = pl.BlockSpec((3 * D, D), lambda i: (0, 0))
    b = pl.BlockSpec((1, D), lambda i: (0, 0))
    return pl.pallas_call(
        _upd_body,
        grid=(_G,),
        in_specs=[row, row, aspec, aspec, aspec, aspec, cnt, w, b, w, b],
        out_specs=[row, row],
        out_shape=[jax.ShapeDtypeStruct((NP, D), _f32)] * 2,
    )(cp, vp, a_pvc, a_nvc, a_pcv, a_ncv, cnt8, wc, bc, wv, bv)


def kernel(v_size, c_size, v_edge_index, c_edge_index, p_edge_index,
           n_edge_index, v_emb, c_emb,
           pv2c_W1, pv2c_b1, pv2c_W2, pv2c_b2,
           nv2c_W1, nv2c_b1, nv2c_W2, nv2c_b2,
           pc2v_W1, pc2v_b1, pc2v_W2, pc2v_b2,
           nc2v_W1, nc2v_b1, nc2v_W2, nc2v_b2,
           c_upd_W, c_upd_b, v_upd_W, v_upd_b):
    pad_t = jnp.full((8,), V, _i32)
    v_ei = jnp.concatenate([v_edge_index, pad_t])
    c_ei = jnp.concatenate([c_edge_index, pad_t])
    pad_e = jnp.full((ROWS_ALLOC * CH - EP,), E, _i32)
    pe = jnp.concatenate([p_edge_index, pad_e]).reshape(ROWS_ALLOC, CH)
    ne = jnp.concatenate([n_edge_index, pad_e]).reshape(ROWS_ALLOC, CH)

    pv, pc, nv, nc, cnt = _sc_prep(v_ei, c_ei, pe, ne)
    pv, pc, nv, nc = (x.reshape(EPAD) for x in (pv, pc, nv, nc))
    cnt8 = cnt.reshape(NC * 4, NP)

    zpad = jnp.zeros((NP - V, D), _f32)
    vp = jnp.concatenate([v_emb, zpad])
    cp = jnp.concatenate([c_emb, zpad])

    ws = (pv2c_W1, pv2c_b1.reshape(1, D), pv2c_W2, pv2c_b2.reshape(1, D),
          nv2c_W1, nv2c_b1.reshape(1, D), nv2c_W2, nv2c_b2.reshape(1, D),
          pc2v_W1, pc2v_b1.reshape(1, D), pc2v_W2, pc2v_b2.reshape(1, D),
          nc2v_W1, nc2v_b1.reshape(1, D), nc2v_W2, nc2v_b2.reshape(1, D))
    bc = c_upd_b.reshape(1, D)
    bv = v_upd_b.reshape(1, D)

    v_list = [vp]
    c_list = [cp]
    for _ in range(ITERS):
        m_pv, m_nv, m_pc, m_nc = _tc_msg(vp, cp, cnt8, *ws)
        a_pvc, a_nvc, a_pcv, a_ncv = _sc_aggr(m_pv, m_nv, m_pc, m_nc,
                                              pv, pc, nv, nc)
        cp, vp = _tc_upd(cp, vp, a_pvc, a_nvc, a_pcv, a_ncv, cnt8,
                         c_upd_W, bc, v_upd_W, bv)
        v_list.append(vp)
        c_list.append(cp)

    v_out = jnp.stack([x[:V] for x in v_list])
    c_out = jnp.stack([x[:V] for x in c_list])
    return (v_out, c_out)


# speedup vs baseline: 4.2231x; 4.2231x over previous
import functools

import jax
import jax.numpy as jnp
from jax import lax
from jax.experimental import pallas as pl
from jax.experimental.pallas import tpu as pltpu
from jax.experimental.pallas import tpu_sc as plsc

V = 10000
D = 128
E = 600000
EP = 300000
NP = 10240
NC = 2
NS = 16
NW = NC * NS
CH = 128
PCH0, PCH1 = 72, 80
ACH0, ACH1 = 72, 80
PCH_MAX = max(PCH0, PCH1)
NCH_MAX = max(ACH0, ACH1)
ROWS = NS * (PCH0 + PCH1)
EPAD = ROWS * CH
ROWS_ALLOC = ROWS
PADT = 2048
RPT = NP // NS
ITERS = 2
_f32 = jnp.float32
_i32 = jnp.int32

_mesh = plsc.VectorSubcoreMesh(core_axis_name="c", subcore_axis_name="s")


@functools.partial(
    pl.kernel,
    out_type=(
        jax.ShapeDtypeStruct((ROWS_ALLOC, CH), _i32),
        jax.ShapeDtypeStruct((ROWS_ALLOC, CH), _i32),
        jax.ShapeDtypeStruct((ROWS_ALLOC, CH), _i32),
        jax.ShapeDtypeStruct((ROWS_ALLOC, CH), _i32),
        jax.ShapeDtypeStruct((NC, 4, NP), _f32),
    ),
    mesh=_mesh,
    scratch_types=(
        pltpu.VMEM((PCH_MAX, CH), _i32),
        pltpu.VMEM((PCH_MAX, CH), _i32),
        pltpu.VMEM((PCH_MAX, CH), _i32),
        pltpu.VMEM((CH,), _f32),
        pltpu.VMEM((RPT,), _f32),
        pltpu.VMEM_SHARED((NP,), _f32),
        pltpu.VMEM_SHARED((NP,), _f32),
        pltpu.VMEM_SHARED((NP,), _f32),
        pltpu.VMEM_SHARED((NP,), _f32),
        tuple(pltpu.SemaphoreType.DMA for _ in range(4)),
        pltpu.SemaphoreType.DMA,
    ),
)
def _sc_prep(v_ei, c_ei, p_idx, n_idx,
             pv_out, pc_out, nv_out, nc_out, cnt_out,
             ebuf, vall, call_, ones, zrows, h_pv, h_pc, h_nv, h_nc, gs, sh):
    cid = lax.axis_index("c")
    sid = lax.axis_index("s")
    pch = PCH0 + cid * (PCH1 - PCH0)
    rb = pl.multiple_of(cid * NS * PCH0 + sid * pch, 8)
    for q in range(CH // 16):
        ones[pl.ds(q * 16, 16)] = jnp.ones((16,), _f32)

    @pl.loop(0, RPT // 16)
    def _(q):
        zrows[pl.ds(q * 16, 16)] = jnp.zeros((16,), _f32)

    for h in (h_pv, h_pc, h_nv, h_nc):
        pltpu.sync_copy(zrows, h.at[pl.ds(sid * RPT, RPT)])
    plsc.subcore_barrier()

    def run(eidx, v_out, c_out, hv, hc):
        pltpu.sync_copy(eidx.at[pl.ds(rb, PCH_MAX)], ebuf)

        def fire(j, b):
            pltpu.async_copy(v_ei.at[ebuf.at[j]], vall.at[j], gs[b])
            pltpu.async_copy(c_ei.at[ebuf.at[j]], call_.at[j], gs[b])

        def drain(b):
            pltpu.make_async_copy(v_ei.at[ebuf.at[0]], vall.at[0],
                                  gs[b]).wait()
            pltpu.make_async_copy(c_ei.at[ebuf.at[0]], call_.at[0],
                                  gs[b]).wait()

        for b in range(3):
            fire(b, b)

        @pl.loop(0, pch // 4)
        def _(t):
            for b in range(4):
                j = 4 * t + b

                @pl.when(j + 3 < pch)
                def _():
                    fire(j + 3, (b + 3) % 4)

                drain(b)
                pltpu.async_copy(ones, hv.at[vall.at[j]], sh, add=True)
                pltpu.async_copy(ones, hc.at[call_.at[j]], sh, add=True)

        @pl.loop(0, 2 * pch)
        def _(j):
            pltpu.make_async_copy(ones, hv.at[vall.at[0]], sh).wait()

        pltpu.sync_copy(vall.at[pl.ds(0, PCH0)], v_out.at[pl.ds(rb, PCH0)])
        pltpu.sync_copy(call_.at[pl.ds(0, PCH0)], c_out.at[pl.ds(rb, PCH0)])

        @pl.when(pch > PCH0)
        def _():
            rb2 = pl.multiple_of(rb + PCH0, 8)
            pltpu.sync_copy(vall.at[pl.ds(PCH0, PCH_MAX - PCH0)],
                            v_out.at[pl.ds(rb2, PCH_MAX - PCH0)])
            pltpu.sync_copy(call_.at[pl.ds(PCH0, PCH_MAX - PCH0)],
                            c_out.at[pl.ds(rb2, PCH_MAX - PCH0)])

    run(p_idx, pv_out, pc_out, h_pv, h_pc)
    run(n_idx, nv_out, nc_out, h_nv, h_nc)
    plsc.subcore_barrier()
    for a, h in enumerate((h_pv, h_pc, h_nv, h_nc)):
        pltpu.sync_copy(h.at[pl.ds(sid * RPT, RPT)],
                        cnt_out.at[cid, a, pl.ds(sid * RPT, RPT)])


@functools.partial(
    pl.kernel,
    out_type=tuple(jax.ShapeDtypeStruct((NC, NP, D), _f32) for _ in range(4)),
    mesh=_mesh,
    scratch_types=(
        tuple(pltpu.VMEM((CH,), _i32) for _ in range(2)),
        tuple(pltpu.VMEM((CH,), _i32) for _ in range(2)),
        tuple(pltpu.VMEM((CH, D), _f32) for _ in range(2)),
        pltpu.VMEM((16, D), _f32),
        pltpu.VMEM_SHARED((NP, D), _f32),
        tuple(pltpu.SemaphoreType.DMA for _ in range(2)),
        tuple(pltpu.SemaphoreType.DMA for _ in range(2)),
    ),
)
def _sc_aggr(m_pv, m_nv, m_pc, m_nc, pv, pc, nv, nc,
             o_pvc, o_nvc, o_pcv, o_ncv,
             sbuf, dbuf, rows, zblk, acc, si, gs):
    cid = lax.axis_index("c")
    sid = lax.axis_index("s")
    nch = ACH0 + cid * (ACH1 - ACH0)
    eb = pl.multiple_of((cid * NS * ACH0 + sid * nch) * CH, 1024)

    @pl.loop(0, 16)
    def _(r):
        for q in range(D // 16):
            zblk[r, pl.ds(q * 16, 16)] = jnp.zeros((16,), _f32)

    jobs = ((m_pv, pv, pc, o_pvc),
            (m_nv, nv, nc, o_nvc),
            (m_pc, pc, pv, o_pcv),
            (m_nc, nc, nv, o_ncv))
    for tab, src, dst, out in jobs:
        @pl.loop(0, RPT // 16)
        def _(b):
            pltpu.sync_copy(zblk, acc.at[pl.ds(sid * RPT + b * 16, 16)])
        plsc.subcore_barrier()

        def startidx(j, b):
            pltpu.async_copy(src.at[pl.ds(eb + j * CH, CH)], sbuf[b], si[b])
            pltpu.async_copy(dst.at[pl.ds(eb + j * CH, CH)], dbuf[b], si[b])

        def startgather(b):
            pltpu.make_async_copy(src.at[pl.ds(0, CH)], sbuf[b],
                                  si[b]).wait()
            pltpu.make_async_copy(dst.at[pl.ds(0, CH)], dbuf[b],
                                  si[b]).wait()
            pltpu.async_copy(tab.at[sbuf[b]], rows[b], gs[b])

        def finish(b):
            pltpu.make_async_copy(tab.at[sbuf[b]], rows[b], gs[b]).wait()
            pltpu.sync_copy(rows[b], acc.at[dbuf[b]], add=True)

        startidx(0, 0)
        startidx(1, 1)
        startgather(0)

        @pl.loop(0, nch // 2)
        def _(t):
            for b in range(2):
                j = 2 * t + b
                o = (b + 1) % 2

                @pl.when(j + 1 < nch)
                def _():
                    startgather(o)

                finish(b)

                @pl.when(j + 2 < nch)
                def _():
                    startidx(j + 2, b)

        plsc.subcore_barrier()
        pltpu.sync_copy(acc.at[pl.ds(sid * RPT, RPT)],
                        out.at[cid, pl.ds(sid * RPT, RPT)])
        plsc.subcore_barrier()


_BLK = 128
_G = NP // _BLK


def _dot(a, b):
    return jnp.dot(a, b, preferred_element_type=_f32,
                   precision=lax.Precision.HIGHEST)


def _msg_body(v_ref, c_ref, cnt_ref,
              pw1, pb1, pw2, pb2, nw1, nb1, nw2, nb2,
              qw1, qb1, qw2, qb2, rw1, rb1, rw2, rb2,
              o_pv, o_nv, o_pc, o_nc):
    cnt = cnt_ref[...]

    def scale(a):
        return lax.rsqrt(jnp.maximum(cnt[a] + cnt[4 + a], 1.0))

    def mlp(x, w1, b1, w2, b2):
        h = jnp.maximum(_dot(x, w1[...]) + b1[...], 0.0)
        return _dot(h, w2[...]) + b2[...]

    xv = v_ref[...]
    xc = c_ref[...]
    o_pv[...] = mlp(xv, pw1, pb1, pw2, pb2) * scale(0)[:, None]
    o_nv[...] = mlp(xv, nw1, nb1, nw2, nb2) * scale(2)[:, None]
    o_pc[...] = mlp(xc, qw1, qb1, qw2, qb2) * scale(1)[:, None]
    o_nc[...] = mlp(xc, rw1, rb1, rw2, rb2) * scale(3)[:, None]


def _tc_msg(vp, cp, cnt8, *ws):
    row = pl.BlockSpec((_BLK, D), lambda i: (i, 0))
    cnt = pl.BlockSpec((8, _BLK), lambda i: (0, i))
    w = pl.BlockSpec((D, D), lambda i: (0, 0))
    b = pl.BlockSpec((1, D), lambda i: (0, 0))
    return pl.pallas_call(
        _msg_body,
        grid=(_G,),
        in_specs=[row, row, cnt] + [w, b, w, b] * 4,
        out_specs=[row] * 4,
        out_shape=[jax.ShapeDtypeStruct((NP, D), _f32)] * 4,
    )(vp, cp, cnt8, *ws)


def _upd_body(c_ref, v_ref, a0, a1, a2, a3, cnt_ref,
              wc, bc, wv, bv, oc, ov):
    cnt = cnt_ref[...]

    def scale(a):
        return lax.rsqrt(jnp.maximum(cnt[a] + cnt[4 + a], 1.0))

    def agg(aref, a):
        x = aref[...]
        return (x[0] + x[1]) * scale(a)[:, None]

    wcm = wc[...]
    wvm = wv[...]
    oc[...] = (_dot(c_ref[...], wcm[0:D]) + _dot(agg(a0, 1), wcm[D:2 * D])
               + _dot(agg(a1, 3), wcm[2 * D:3 * D]) + bc[...])
    ov[...] = (_dot(v_ref[...], wvm[0:D]) + _dot(agg(a2, 0), wvm[D:2 * D])
               + _dot(agg(a3, 2), wvm[2 * D:3 * D]) + bv[...])


def _tc_upd(cp, vp, a_pvc, a_nvc, a_pcv, a_ncv, cnt8, wc, bc, wv, bv):
    row = pl.BlockSpec((_BLK, D), lambda i: (i, 0))
    aspec = pl.BlockSpec((NC, _BLK, D), lambda i: (0, i, 0))
    cnt = pl.BlockSpec((8, _BLK), lambda i: (0, i))
    w = pl.BlockSpec((3 * D, D), lambda i: (0, 0))
    b = pl.BlockSpec((1, D), lambda i: (0, 0))
    return pl.pallas_call(
        _upd_body,
        grid=(_G,),
        in_specs=[row, row, aspec, aspec, aspec, aspec, cnt, w, b, w, b],
        out_specs=[row, row],
        out_shape=[jax.ShapeDtypeStruct((NP, D), _f32)] * 2,
    )(cp, vp, a_pvc, a_nvc, a_pcv, a_ncv, cnt8, wc, bc, wv, bv)


def kernel(v_size, c_size, v_edge_index, c_edge_index, p_edge_index,
           n_edge_index, v_emb, c_emb,
           pv2c_W1, pv2c_b1, pv2c_W2, pv2c_b2,
           nv2c_W1, nv2c_b1, nv2c_W2, nv2c_b2,
           pc2v_W1, pc2v_b1, pc2v_W2, pc2v_b2,
           nc2v_W1, nc2v_b1, nc2v_W2, nc2v_b2,
           c_upd_W, c_upd_b, v_upd_W, v_upd_b):
    pad_t = V + (jnp.arange(PADT, dtype=_i32) % (NP - V))
    v_ei = jnp.concatenate([v_edge_index, pad_t])
    c_ei = jnp.concatenate([c_edge_index, pad_t])
    pad_e = E + (jnp.arange(ROWS_ALLOC * CH - EP, dtype=_i32) % PADT)
    pe = jnp.concatenate([p_edge_index, pad_e]).reshape(ROWS_ALLOC, CH)
    ne = jnp.concatenate([n_edge_index, pad_e]).reshape(ROWS_ALLOC, CH)

    pv, pc, nv, nc, cnt = _sc_prep(v_ei, c_ei, pe, ne)
    pv, pc, nv, nc = (x.reshape(EPAD) for x in (pv, pc, nv, nc))
    cnt8 = cnt.reshape(NC * 4, NP)

    zpad = jnp.zeros((NP - V, D), _f32)
    vp = jnp.concatenate([v_emb, zpad])
    cp = jnp.concatenate([c_emb, zpad])

    ws = (pv2c_W1, pv2c_b1.reshape(1, D), pv2c_W2, pv2c_b2.reshape(1, D),
          nv2c_W1, nv2c_b1.reshape(1, D), nv2c_W2, nv2c_b2.reshape(1, D),
          pc2v_W1, pc2v_b1.reshape(1, D), pc2v_W2, pc2v_b2.reshape(1, D),
          nc2v_W1, nc2v_b1.reshape(1, D), nc2v_W2, nc2v_b2.reshape(1, D))
    bc = c_upd_b.reshape(1, D)
    bv = v_upd_b.reshape(1, D)

    v_list = [vp]
    c_list = [cp]
    for _ in range(ITERS):
        m_pv, m_nv, m_pc, m_nc = _tc_msg(vp, cp, cnt8, *ws)
        a_pvc, a_nvc, a_pcv, a_ncv = _sc_aggr(m_pv, m_nv, m_pc, m_nc,
                                              pv, pc, nv, nc)
        cp, vp = _tc_upd(cp, vp, a_pvc, a_nvc, a_pcv, a_ncv, cnt8,
                         c_upd_W, bc, v_upd_W, bv)
        v_list.append(vp)
        c_list.append(cp)

    v_out = jnp.stack([x[:V] for x in v_list])
    c_out = jnp.stack([x[:V] for x in c_list])
    return (v_out, c_out)


# speedup vs baseline: 4.6215x; 1.0943x over previous
import functools

import jax
import jax.numpy as jnp
from jax import lax
from jax.experimental import pallas as pl
from jax.experimental.pallas import tpu as pltpu
from jax.experimental.pallas import tpu_sc as plsc

V = 10000
D = 128
E = 600000
EP = 300000
NP = 10240
NC = 2
NS = 16
NW = NC * NS
CH = 128
PCH0, PCH1 = 72, 80
ACH0, ACH1 = 72, 80
PCH_MAX = max(PCH0, PCH1)
NCH_MAX = max(ACH0, ACH1)
ROWS = NS * (PCH0 + PCH1)
EPAD = ROWS * CH
ROWS_ALLOC = ROWS
PADT = 2048
RPT = NP // NS
ITERS = 2
_f32 = jnp.float32
_i32 = jnp.int32

_mesh = plsc.VectorSubcoreMesh(core_axis_name="c", subcore_axis_name="s")


@functools.partial(
    pl.kernel,
    out_type=(
        jax.ShapeDtypeStruct((ROWS_ALLOC, CH), _i32),
        jax.ShapeDtypeStruct((ROWS_ALLOC, CH), _i32),
        jax.ShapeDtypeStruct((ROWS_ALLOC, CH), _i32),
        jax.ShapeDtypeStruct((ROWS_ALLOC, CH), _i32),
        jax.ShapeDtypeStruct((NC, 4, NP), _f32),
    ),
    mesh=_mesh,
    scratch_types=(
        pltpu.VMEM((PCH_MAX, CH), _i32),
        pltpu.VMEM((PCH_MAX, CH), _i32),
        pltpu.VMEM((PCH_MAX, CH), _i32),
        pltpu.VMEM((CH,), _f32),
        pltpu.VMEM((RPT,), _f32),
        pltpu.VMEM_SHARED((NP,), _f32),
        pltpu.VMEM_SHARED((NP,), _f32),
        pltpu.VMEM_SHARED((NP,), _f32),
        pltpu.VMEM_SHARED((NP,), _f32),
        tuple(pltpu.SemaphoreType.DMA for _ in range(4)),
        pltpu.SemaphoreType.DMA,
    ),
)
def _sc_prep(v_ei, c_ei, p_idx, n_idx,
             pv_out, pc_out, nv_out, nc_out, cnt_out,
             ebuf, vall, call_, ones, zrows, h_pv, h_pc, h_nv, h_nc, gs, sh):
    cid = lax.axis_index("c")
    sid = lax.axis_index("s")
    pch = PCH0 + cid * (PCH1 - PCH0)
    rb = pl.multiple_of(cid * NS * PCH0 + sid * pch, 8)
    for q in range(CH // 16):
        ones[pl.ds(q * 16, 16)] = jnp.ones((16,), _f32)

    @pl.loop(0, RPT // 16)
    def _(q):
        zrows[pl.ds(q * 16, 16)] = jnp.zeros((16,), _f32)

    for h in (h_pv, h_pc, h_nv, h_nc):
        pltpu.sync_copy(zrows, h.at[pl.ds(sid * RPT, RPT)])
    plsc.subcore_barrier()

    def run(eidx, v_out, c_out, hv, hc):
        pltpu.sync_copy(eidx.at[pl.ds(rb, PCH_MAX)], ebuf)

        def fire(j, b):
            pltpu.async_copy(v_ei.at[ebuf.at[j]], vall.at[j], gs[b])
            pltpu.async_copy(c_ei.at[ebuf.at[j]], call_.at[j], gs[b])

        def drain(b):
            pltpu.make_async_copy(v_ei.at[ebuf.at[0]], vall.at[0],
                                  gs[b]).wait()
            pltpu.make_async_copy(c_ei.at[ebuf.at[0]], call_.at[0],
                                  gs[b]).wait()

        for b in range(3):
            fire(b, b)

        @pl.loop(0, pch // 4)
        def _(t):
            for b in range(4):
                j = 4 * t + b

                @pl.when(j + 3 < pch)
                def _():
                    fire(j + 3, (b + 3) % 4)

                drain(b)
                pltpu.async_copy(ones, hv.at[vall.at[j]], sh, add=True)
                pltpu.async_copy(ones, hc.at[call_.at[j]], sh, add=True)

        @pl.loop(0, 2 * pch)
        def _(j):
            pltpu.make_async_copy(ones, hv.at[vall.at[0]], sh).wait()

        pltpu.sync_copy(vall.at[pl.ds(0, PCH0)], v_out.at[pl.ds(rb, PCH0)])
        pltpu.sync_copy(call_.at[pl.ds(0, PCH0)], c_out.at[pl.ds(rb, PCH0)])

        @pl.when(pch > PCH0)
        def _():
            rb2 = pl.multiple_of(rb + PCH0, 8)
            pltpu.sync_copy(vall.at[pl.ds(PCH0, PCH_MAX - PCH0)],
                            v_out.at[pl.ds(rb2, PCH_MAX - PCH0)])
            pltpu.sync_copy(call_.at[pl.ds(PCH0, PCH_MAX - PCH0)],
                            c_out.at[pl.ds(rb2, PCH_MAX - PCH0)])

    run(p_idx, pv_out, pc_out, h_pv, h_pc)
    run(n_idx, nv_out, nc_out, h_nv, h_nc)
    plsc.subcore_barrier()
    for a, h in enumerate((h_pv, h_pc, h_nv, h_nc)):
        pltpu.sync_copy(h.at[pl.ds(sid * RPT, RPT)],
                        cnt_out.at[cid, a, pl.ds(sid * RPT, RPT)])


@functools.partial(
    pl.kernel,
    out_type=tuple(jax.ShapeDtypeStruct((NC, NP, D), _f32) for _ in range(4)),
    mesh=_mesh,
    scratch_types=(
        tuple(pltpu.VMEM((CH,), _i32) for _ in range(4)),
        tuple(pltpu.VMEM((CH,), _i32) for _ in range(4)),
        tuple(pltpu.VMEM((CH, D), _f32) for _ in range(2)),
        pltpu.VMEM((16, D), _f32),
        pltpu.VMEM_SHARED((NP, D), _f32),
        tuple(pltpu.SemaphoreType.DMA for _ in range(4)),
        tuple(pltpu.SemaphoreType.DMA for _ in range(2)),
        tuple(pltpu.SemaphoreType.DMA for _ in range(2)),
    ),
)
def _sc_aggr(m_pv, m_nv, m_pc, m_nc, pv, pc, nv, nc,
             o_pvc, o_nvc, o_pcv, o_ncv,
             sbuf, dbuf, rows, zblk, acc, si, gs, ss):
    cid = lax.axis_index("c")
    sid = lax.axis_index("s")
    nch = ACH0 + cid * (ACH1 - ACH0)
    eb = pl.multiple_of((cid * NS * ACH0 + sid * nch) * CH, 1024)

    @pl.loop(0, 16)
    def _(r):
        for q in range(D // 16):
            zblk[r, pl.ds(q * 16, 16)] = jnp.zeros((16,), _f32)

    jobs = ((m_pv, pv, pc, o_pvc),
            (m_nv, nv, nc, o_nvc),
            (m_pc, pc, pv, o_pcv),
            (m_nc, nc, nv, o_ncv))
    for tab, src, dst, out in jobs:
        @pl.loop(0, RPT // 16)
        def _(b):
            pltpu.sync_copy(zblk, acc.at[pl.ds(sid * RPT + b * 16, 16)])
        plsc.subcore_barrier()

        def startidx(j, q):
            pltpu.async_copy(src.at[pl.ds(eb + j * CH, CH)], sbuf[q], si[q])
            pltpu.async_copy(dst.at[pl.ds(eb + j * CH, CH)], dbuf[q], si[q])

        def startgather(j, b, q):
            pltpu.make_async_copy(src.at[pl.ds(0, CH)], sbuf[q],
                                  si[q]).wait()
            pltpu.make_async_copy(dst.at[pl.ds(0, CH)], dbuf[q],
                                  si[q]).wait()

            @pl.when(j >= 2)
            def _():
                pltpu.make_async_copy(rows[b], acc.at[dbuf[0]],
                                      ss[b]).wait()

            pltpu.async_copy(tab.at[sbuf[q]], rows[b], gs[b])

        def finish(b, q):
            pltpu.make_async_copy(tab.at[sbuf[q]], rows[b], gs[b]).wait()
            pltpu.async_copy(rows[b], acc.at[dbuf[q]], ss[b], add=True)

        startidx(0, 0)
        startidx(1, 1)
        startgather(0, 0, 0)

        @pl.loop(0, nch // 4)
        def _(t):
            for b in range(4):
                j = 4 * t + b

                @pl.when(j + 1 < nch)
                def _():
                    startgather(j + 1, (b + 1) % 2, (b + 1) % 4)

                finish(b % 2, b)

                @pl.when(j + 2 < nch)
                def _():
                    startidx(j + 2, (b + 2) % 4)

        for b in range(2):
            pltpu.make_async_copy(rows[b], acc.at[dbuf[0]], ss[b]).wait()
        plsc.subcore_barrier()
        pltpu.sync_copy(acc.at[pl.ds(sid * RPT, RPT)],
                        out.at[cid, pl.ds(sid * RPT, RPT)])
        plsc.subcore_barrier()


_BLK = 128
_G = NP // _BLK


def _dot(a, b):
    return jnp.dot(a, b, preferred_element_type=_f32,
                   precision=lax.Precision.HIGHEST)


def _msg_body(v_ref, c_ref, cnt_ref,
              pw1, pb1, pw2, pb2, nw1, nb1, nw2, nb2,
              qw1, qb1, qw2, qb2, rw1, rb1, rw2, rb2,
              o_pv, o_nv, o_pc, o_nc):
    cnt = cnt_ref[...]

    def scale(a):
        return lax.rsqrt(jnp.maximum(cnt[a] + cnt[4 + a], 1.0))

    def mlp(x, w1, b1, w2, b2):
        h = jnp.maximum(_dot(x, w1[...]) + b1[...], 0.0)
        return _dot(h, w2[...]) + b2[...]

    xv = v_ref[...]
    xc = c_ref[...]
    o_pv[...] = mlp(xv, pw1, pb1, pw2, pb2) * scale(0)[:, None]
    o_nv[...] = mlp(xv, nw1, nb1, nw2, nb2) * scale(2)[:, None]
    o_pc[...] = mlp(xc, qw1, qb1, qw2, qb2) * scale(1)[:, None]
    o_nc[...] = mlp(xc, rw1, rb1, rw2, rb2) * scale(3)[:, None]


def _tc_msg(vp, cp, cnt8, *ws):
    row = pl.BlockSpec((_BLK, D), lambda i: (i, 0))
    cnt = pl.BlockSpec((8, _BLK), lambda i: (0, i))
    w = pl.BlockSpec((D, D), lambda i: (0, 0))
    b = pl.BlockSpec((1, D), lambda i: (0, 0))
    return pl.pallas_call(
        _msg_body,
        grid=(_G,),
        in_specs=[row, row, cnt] + [w, b, w, b] * 4,
        out_specs=[row] * 4,
        out_shape=[jax.ShapeDtypeStruct((NP, D), _f32)] * 4,
    )(vp, cp, cnt8, *ws)


def _upd_body(c_ref, v_ref, a0, a1, a2, a3, cnt_ref,
              wc, bc, wv, bv, oc, ov):
    cnt = cnt_ref[...]

    def scale(a):
        return lax.rsqrt(jnp.maximum(cnt[a] + cnt[4 + a], 1.0))

    def agg(aref, a):
        x = aref[...]
        return (x[0] + x[1]) * scale(a)[:, None]

    wcm = wc[...]
    wvm = wv[...]
    oc[...] = (_dot(c_ref[...], wcm[0:D]) + _dot(agg(a0, 1), wcm[D:2 * D])
               + _dot(agg(a1, 3), wcm[2 * D:3 * D]) + bc[...])
    ov[...] = (_dot(v_ref[...], wvm[0:D]) + _dot(agg(a2, 0), wvm[D:2 * D])
               + _dot(agg(a3, 2), wvm[2 * D:3 * D]) + bv[...])


def _tc_upd(cp, vp, a_pvc, a_nvc, a_pcv, a_ncv, cnt8, wc, bc, wv, bv):
    row = pl.BlockSpec((_BLK, D), lambda i: (i, 0))
    aspec = pl.BlockSpec((NC, _BLK, D), lambda i: (0, i, 0))
    cnt = pl.BlockSpec((8, _BLK), lambda i: (0, i))
    w = pl.BlockSpec((3 * D, D), lambda i: (0, 0))
    b = pl.BlockSpec((1, D), lambda i: (0, 0))
    return pl.pallas_call(
        _upd_body,
        grid=(_G,),
        in_specs=[row, row, aspec, aspec, aspec, aspec, cnt, w, b, w, b],
        out_specs=[row, row],
        out_shape=[jax.ShapeDtypeStruct((NP, D), _f32)] * 2,
    )(cp, vp, a_pvc, a_nvc, a_pcv, a_ncv, cnt8, wc, bc, wv, bv)


def kernel(v_size, c_size, v_edge_index, c_edge_index, p_edge_index,
           n_edge_index, v_emb, c_emb,
           pv2c_W1, pv2c_b1, pv2c_W2, pv2c_b2,
           nv2c_W1, nv2c_b1, nv2c_W2, nv2c_b2,
           pc2v_W1, pc2v_b1, pc2v_W2, pc2v_b2,
           nc2v_W1, nc2v_b1, nc2v_W2, nc2v_b2,
           c_upd_W, c_upd_b, v_upd_W, v_upd_b):
    pad_t = V + (jnp.arange(PADT, dtype=_i32) % (NP - V))
    v_ei = jnp.concatenate([v_edge_index, pad_t])
    c_ei = jnp.concatenate([c_edge_index, pad_t])
    pad_e = E + (jnp.arange(ROWS_ALLOC * CH - EP, dtype=_i32) % PADT)
    pe = jnp.concatenate([p_edge_index, pad_e]).reshape(ROWS_ALLOC, CH)
    ne = jnp.concatenate([n_edge_index, pad_e]).reshape(ROWS_ALLOC, CH)

    pv, pc, nv, nc, cnt = _sc_prep(v_ei, c_ei, pe, ne)
    pv, pc, nv, nc = (x.reshape(EPAD) for x in (pv, pc, nv, nc))
    cnt8 = cnt.reshape(NC * 4, NP)

    zpad = jnp.zeros((NP - V, D), _f32)
    vp = jnp.concatenate([v_emb, zpad])
    cp = jnp.concatenate([c_emb, zpad])

    ws = (pv2c_W1, pv2c_b1.reshape(1, D), pv2c_W2, pv2c_b2.reshape(1, D),
          nv2c_W1, nv2c_b1.reshape(1, D), nv2c_W2, nv2c_b2.reshape(1, D),
          pc2v_W1, pc2v_b1.reshape(1, D), pc2v_W2, pc2v_b2.reshape(1, D),
          nc2v_W1, nc2v_b1.reshape(1, D), nc2v_W2, nc2v_b2.reshape(1, D))
    bc = c_upd_b.reshape(1, D)
    bv = v_upd_b.reshape(1, D)

    v_list = [vp]
    c_list = [cp]
    for _ in range(ITERS):
        m_pv, m_nv, m_pc, m_nc = _tc_msg(vp, cp, cnt8, *ws)
        a_pvc, a_nvc, a_pcv, a_ncv = _sc_aggr(m_pv, m_nv, m_pc, m_nc,
                                              pv, pc, nv, nc)
        cp, vp = _tc_upd(cp, vp, a_pvc, a_nvc, a_pcv, a_ncv, cnt8,
                         c_upd_W, bc, v_upd_W, bv)
        v_list.append(vp)
        c_list.append(cp)

    v_out = jnp.stack([x[:V] for x in v_list])
    c_out = jnp.stack([x[:V] for x in c_list])
    return (v_out, c_out)


# speedup vs baseline: 4.8966x; 1.0595x over previous
import functools

import jax
import jax.numpy as jnp
from jax import lax
from jax.experimental import pallas as pl
from jax.experimental.pallas import tpu as pltpu
from jax.experimental.pallas import tpu_sc as plsc

V = 10000
D = 128
E = 600000
EP = 300000
NP = 10240
NC = 2
NS = 16
NW = NC * NS
CH = 128
PCH0, PCH1 = 72, 80
ACH0, ACH1 = 72, 80
PCH_MAX = max(PCH0, PCH1)
NCH_MAX = max(ACH0, ACH1)
ROWS = NS * (PCH0 + PCH1)
EPAD = ROWS * CH
ROWS_ALLOC = ROWS
PADT = 2048
RPT = NP // NS
ITERS = 2
_f32 = jnp.float32
_i32 = jnp.int32

_mesh = plsc.VectorSubcoreMesh(core_axis_name="c", subcore_axis_name="s")


@functools.partial(
    pl.kernel,
    out_type=(
        jax.ShapeDtypeStruct((ROWS_ALLOC, CH), _i32),
        jax.ShapeDtypeStruct((ROWS_ALLOC, CH), _i32),
        jax.ShapeDtypeStruct((ROWS_ALLOC, CH), _i32),
        jax.ShapeDtypeStruct((ROWS_ALLOC, CH), _i32),
        jax.ShapeDtypeStruct((NC, 4, NP), _f32),
    ),
    mesh=_mesh,
    scratch_types=(
        pltpu.VMEM((PCH_MAX, CH), _i32),
        pltpu.VMEM((PCH_MAX, CH), _i32),
        pltpu.VMEM((PCH_MAX, CH), _i32),
        pltpu.VMEM((CH,), _f32),
        pltpu.VMEM((RPT,), _f32),
        pltpu.VMEM_SHARED((NP,), _f32),
        pltpu.VMEM_SHARED((NP,), _f32),
        pltpu.VMEM_SHARED((NP,), _f32),
        pltpu.VMEM_SHARED((NP,), _f32),
        tuple(pltpu.SemaphoreType.DMA for _ in range(4)),
        pltpu.SemaphoreType.DMA,
    ),
)
def _sc_prep(v_ei, c_ei, p_idx, n_idx,
             pv_out, pc_out, nv_out, nc_out, cnt_out,
             ebuf, vall, call_, ones, zrows, h_pv, h_pc, h_nv, h_nc, gs, sh):
    cid = lax.axis_index("c")
    sid = lax.axis_index("s")
    pch = PCH0 + cid * (PCH1 - PCH0)
    rb = pl.multiple_of(cid * NS * PCH0 + sid * pch, 8)
    for q in range(CH // 16):
        ones[pl.ds(q * 16, 16)] = jnp.ones((16,), _f32)

    @pl.loop(0, RPT // 16)
    def _(q):
        zrows[pl.ds(q * 16, 16)] = jnp.zeros((16,), _f32)

    for h in (h_pv, h_pc, h_nv, h_nc):
        pltpu.sync_copy(zrows, h.at[pl.ds(sid * RPT, RPT)])
    plsc.subcore_barrier()

    def run(eidx, v_out, c_out, hv, hc):
        pltpu.sync_copy(eidx.at[pl.ds(rb, PCH_MAX)], ebuf)

        def fire(j, b):
            pltpu.async_copy(v_ei.at[ebuf.at[j]], vall.at[j], gs[b])
            pltpu.async_copy(c_ei.at[ebuf.at[j]], call_.at[j], gs[b])

        def drain(b):
            pltpu.make_async_copy(v_ei.at[ebuf.at[0]], vall.at[0],
                                  gs[b]).wait()
            pltpu.make_async_copy(c_ei.at[ebuf.at[0]], call_.at[0],
                                  gs[b]).wait()

        for b in range(3):
            fire(b, b)

        @pl.loop(0, pch // 4)
        def _(t):
            for b in range(4):
                j = 4 * t + b

                @pl.when(j + 3 < pch)
                def _():
                    fire(j + 3, (b + 3) % 4)

                drain(b)
                pltpu.async_copy(ones, hv.at[vall.at[j]], sh, add=True)
                pltpu.async_copy(ones, hc.at[call_.at[j]], sh, add=True)

        @pl.loop(0, 2 * pch)
        def _(j):
            pltpu.make_async_copy(ones, hv.at[vall.at[0]], sh).wait()

        pltpu.sync_copy(vall.at[pl.ds(0, PCH0)], v_out.at[pl.ds(rb, PCH0)])
        pltpu.sync_copy(call_.at[pl.ds(0, PCH0)], c_out.at[pl.ds(rb, PCH0)])

        @pl.when(pch > PCH0)
        def _():
            rb2 = pl.multiple_of(rb + PCH0, 8)
            pltpu.sync_copy(vall.at[pl.ds(PCH0, PCH_MAX - PCH0)],
                            v_out.at[pl.ds(rb2, PCH_MAX - PCH0)])
            pltpu.sync_copy(call_.at[pl.ds(PCH0, PCH_MAX - PCH0)],
                            c_out.at[pl.ds(rb2, PCH_MAX - PCH0)])

    run(p_idx, pv_out, pc_out, h_pv, h_pc)
    run(n_idx, nv_out, nc_out, h_nv, h_nc)
    plsc.subcore_barrier()
    for a, h in enumerate((h_pv, h_pc, h_nv, h_nc)):
        pltpu.sync_copy(h.at[pl.ds(sid * RPT, RPT)],
                        cnt_out.at[cid, a, pl.ds(sid * RPT, RPT)])


@functools.partial(
    pl.kernel,
    out_type=tuple(jax.ShapeDtypeStruct((NC, NP, D), _f32) for _ in range(4)),
    mesh=_mesh,
    scratch_types=(
        tuple(pltpu.VMEM((CH,), _i32) for _ in range(4)),
        tuple(pltpu.VMEM((CH,), _i32) for _ in range(4)),
        tuple(pltpu.VMEM((CH, D), _f32) for _ in range(2)),
        pltpu.VMEM((16, D), _f32),
        pltpu.VMEM_SHARED((NP, D), _f32),
        tuple(pltpu.SemaphoreType.DMA for _ in range(4)),
        tuple(pltpu.SemaphoreType.DMA for _ in range(2)),
        tuple(pltpu.SemaphoreType.DMA for _ in range(2)),
    ),
)
def _sc_aggr(m_pv, m_nv, m_pc, m_nc, pv, pc, nv, nc,
             o_pvc, o_nvc, o_pcv, o_ncv,
             sbuf, dbuf, rows, zblk, acc, si, gs, ss):
    cid = lax.axis_index("c")
    sid = lax.axis_index("s")
    nch = ACH0 + cid * (ACH1 - ACH0)
    eb = pl.multiple_of((cid * NS * ACH0 + sid * nch) * CH, 1024)

    @pl.loop(0, 16)
    def _(r):
        for q in range(D // 16):
            zblk[r, pl.ds(q * 16, 16)] = jnp.zeros((16,), _f32)

    jobs = ((m_pv, pv, pc, o_pvc),
            (m_nv, nv, nc, o_nvc),
            (m_pc, pc, pv, o_pcv),
            (m_nc, nc, nv, o_ncv))
    for tab, src, dst, out in jobs:
        @pl.loop(0, RPT // 16)
        def _(b):
            pltpu.sync_copy(zblk, acc.at[pl.ds(sid * RPT + b * 16, 16)])
        plsc.subcore_barrier()

        def startidx(j, q):
            pltpu.async_copy(src.at[pl.ds(eb + j * CH, CH)], sbuf[q], si[q])
            pltpu.async_copy(dst.at[pl.ds(eb + j * CH, CH)], dbuf[q], si[q])

        def startgather(j, b, q):
            pltpu.make_async_copy(src.at[pl.ds(0, CH)], sbuf[q],
                                  si[q]).wait()
            pltpu.make_async_copy(dst.at[pl.ds(0, CH)], dbuf[q],
                                  si[q]).wait()

            @pl.when(j >= 2)
            def _():
                pltpu.make_async_copy(rows[b], acc.at[dbuf[0]],
                                      ss[b]).wait()

            pltpu.async_copy(tab.at[sbuf[q]], rows[b], gs[b])

        def finish(b, q):
            pltpu.make_async_copy(tab.at[sbuf[q]], rows[b], gs[b]).wait()
            pltpu.async_copy(rows[b], acc.at[dbuf[q]], ss[b], add=True)

        startidx(0, 0)
        startidx(1, 1)
        startgather(0, 0, 0)

        @pl.loop(0, nch // 4)
        def _(t):
            for b in range(4):
                j = 4 * t + b

                @pl.when(j + 1 < nch)
                def _():
                    startgather(j + 1, (b + 1) % 2, (b + 1) % 4)

                finish(b % 2, b)

                @pl.when(j + 2 < nch)
                def _():
                    startidx(j + 2, (b + 2) % 4)

        for b in range(2):
            pltpu.make_async_copy(rows[b], acc.at[dbuf[0]], ss[b]).wait()
        plsc.subcore_barrier()
        pltpu.sync_copy(acc.at[pl.ds(sid * RPT, RPT)],
                        out.at[cid, pl.ds(sid * RPT, RPT)])
        plsc.subcore_barrier()


_BLK = 128
_G = NP // _BLK


def _dot(a, b):
    return jnp.dot(a, b, preferred_element_type=_f32)


def _msg_body(v_ref, c_ref, cnt_ref,
              pw1, pb1, pw2, pb2, nw1, nb1, nw2, nb2,
              qw1, qb1, qw2, qb2, rw1, rb1, rw2, rb2,
              o_pv, o_nv, o_pc, o_nc):
    cnt = cnt_ref[...]

    def scale(a):
        return lax.rsqrt(jnp.maximum(cnt[a] + cnt[4 + a], 1.0))

    def mlp(x, w1, b1, w2, b2):
        h = jnp.maximum(_dot(x, w1[...]) + b1[...], 0.0)
        return _dot(h, w2[...]) + b2[...]

    xv = v_ref[...]
    xc = c_ref[...]
    o_pv[...] = mlp(xv, pw1, pb1, pw2, pb2) * scale(0)[:, None]
    o_nv[...] = mlp(xv, nw1, nb1, nw2, nb2) * scale(2)[:, None]
    o_pc[...] = mlp(xc, qw1, qb1, qw2, qb2) * scale(1)[:, None]
    o_nc[...] = mlp(xc, rw1, rb1, rw2, rb2) * scale(3)[:, None]


def _tc_msg(vp, cp, cnt8, *ws):
    row = pl.BlockSpec((_BLK, D), lambda i: (i, 0))
    cnt = pl.BlockSpec((8, _BLK), lambda i: (0, i))
    w = pl.BlockSpec((D, D), lambda i: (0, 0))
    b = pl.BlockSpec((1, D), lambda i: (0, 0))
    return pl.pallas_call(
        _msg_body,
        grid=(_G,),
        in_specs=[row, row, cnt] + [w, b, w, b] * 4,
        out_specs=[row] * 4,
        out_shape=[jax.ShapeDtypeStruct((NP, D), _f32)] * 4,
    )(vp, cp, cnt8, *ws)


def _upd_body(c_ref, v_ref, a0, a1, a2, a3, cnt_ref,
              wc, bc, wv, bv, oc, ov):
    cnt = cnt_ref[...]

    def scale(a):
        return lax.rsqrt(jnp.maximum(cnt[a] + cnt[4 + a], 1.0))

    def agg(aref, a):
        x = aref[...]
        return (x[0] + x[1]) * scale(a)[:, None]

    wcm = wc[...]
    wvm = wv[...]
    oc[...] = (_dot(c_ref[...], wcm[0:D]) + _dot(agg(a0, 1), wcm[D:2 * D])
               + _dot(agg(a1, 3), wcm[2 * D:3 * D]) + bc[...])
    ov[...] = (_dot(v_ref[...], wvm[0:D]) + _dot(agg(a2, 0), wvm[D:2 * D])
               + _dot(agg(a3, 2), wvm[2 * D:3 * D]) + bv[...])


def _tc_upd(cp, vp, a_pvc, a_nvc, a_pcv, a_ncv, cnt8, wc, bc, wv, bv):
    row = pl.BlockSpec((_BLK, D), lambda i: (i, 0))
    aspec = pl.BlockSpec((NC, _BLK, D), lambda i: (0, i, 0))
    cnt = pl.BlockSpec((8, _BLK), lambda i: (0, i))
    w = pl.BlockSpec((3 * D, D), lambda i: (0, 0))
    b = pl.BlockSpec((1, D), lambda i: (0, 0))
    return pl.pallas_call(
        _upd_body,
        grid=(_G,),
        in_specs=[row, row, aspec, aspec, aspec, aspec, cnt, w, b, w, b],
        out_specs=[row, row],
        out_shape=[jax.ShapeDtypeStruct((NP, D), _f32)] * 2,
    )(cp, vp, a_pvc, a_nvc, a_pcv, a_ncv, cnt8, wc, bc, wv, bv)


def kernel(v_size, c_size, v_edge_index, c_edge_index, p_edge_index,
           n_edge_index, v_emb, c_emb,
           pv2c_W1, pv2c_b1, pv2c_W2, pv2c_b2,
           nv2c_W1, nv2c_b1, nv2c_W2, nv2c_b2,
           pc2v_W1, pc2v_b1, pc2v_W2, pc2v_b2,
           nc2v_W1, nc2v_b1, nc2v_W2, nc2v_b2,
           c_upd_W, c_upd_b, v_upd_W, v_upd_b):
    pad_t = V + (jnp.arange(PADT, dtype=_i32) % (NP - V))
    v_ei = jnp.concatenate([v_edge_index, pad_t])
    c_ei = jnp.concatenate([c_edge_index, pad_t])
    pad_e = E + (jnp.arange(ROWS_ALLOC * CH - EP, dtype=_i32) % PADT)
    pe = jnp.concatenate([p_edge_index, pad_e]).reshape(ROWS_ALLOC, CH)
    ne = jnp.concatenate([n_edge_index, pad_e]).reshape(ROWS_ALLOC, CH)

    pv, pc, nv, nc, cnt = _sc_prep(v_ei, c_ei, pe, ne)
    pv, pc, nv, nc = (x.reshape(EPAD) for x in (pv, pc, nv, nc))
    cnt8 = cnt.reshape(NC * 4, NP)

    zpad = jnp.zeros((NP - V, D), _f32)
    vp = jnp.concatenate([v_emb, zpad])
    cp = jnp.concatenate([c_emb, zpad])

    ws = (pv2c_W1, pv2c_b1.reshape(1, D), pv2c_W2, pv2c_b2.reshape(1, D),
          nv2c_W1, nv2c_b1.reshape(1, D), nv2c_W2, nv2c_b2.reshape(1, D),
          pc2v_W1, pc2v_b1.reshape(1, D), pc2v_W2, pc2v_b2.reshape(1, D),
          nc2v_W1, nc2v_b1.reshape(1, D), nc2v_W2, nc2v_b2.reshape(1, D))
    bc = c_upd_b.reshape(1, D)
    bv = v_upd_b.reshape(1, D)

    v_list = [vp]
    c_list = [cp]
    for _ in range(ITERS):
        m_pv, m_nv, m_pc, m_nc = _tc_msg(vp, cp, cnt8, *ws)
        a_pvc, a_nvc, a_pcv, a_ncv = _sc_aggr(m_pv, m_nv, m_pc, m_nc,
                                              pv, pc, nv, nc)
        cp, vp = _tc_upd(cp, vp, a_pvc, a_nvc, a_pcv, a_ncv, cnt8,
                         c_upd_W, bc, v_upd_W, bv)
        v_list.append(vp)
        c_list.append(cp)

    v_out = jnp.stack([x[:V] for x in v_list])
    c_out = jnp.stack([x[:V] for x in c_list])
    return (v_out, c_out)


# speedup vs baseline: 4.9682x; 1.0146x over previous
import functools

import jax
import jax.numpy as jnp
from jax import lax
from jax.experimental import pallas as pl
from jax.experimental.pallas import tpu as pltpu
from jax.experimental.pallas import tpu_sc as plsc

V = 10000
D = 128
E = 600000
EP = 300000
NP = 10240
NC = 2
NS = 16
NW = NC * NS
CH = 128
PCH0, PCH1 = 72, 80
ACH0, ACH1 = 72, 80
PCH_MAX = max(PCH0, PCH1)
NCH_MAX = max(ACH0, ACH1)
ROWS = NS * (PCH0 + PCH1)
EPAD = ROWS * CH
ROWS_ALLOC = ROWS
PADT = 2048
RPT = NP // NS
ITERS = 2
_f32 = jnp.float32
_i32 = jnp.int32

_mesh = plsc.VectorSubcoreMesh(core_axis_name="c", subcore_axis_name="s")


@functools.partial(
    pl.kernel,
    out_type=(
        jax.ShapeDtypeStruct((ROWS_ALLOC, CH), _i32),
        jax.ShapeDtypeStruct((ROWS_ALLOC, CH), _i32),
        jax.ShapeDtypeStruct((ROWS_ALLOC, CH), _i32),
        jax.ShapeDtypeStruct((ROWS_ALLOC, CH), _i32),
        jax.ShapeDtypeStruct((NC, 4, NP), _f32),
    ),
    mesh=_mesh,
    scratch_types=(
        pltpu.VMEM((PCH_MAX, CH), _i32),
        pltpu.VMEM((PCH_MAX, CH), _i32),
        pltpu.VMEM((PCH_MAX, CH), _i32),
        pltpu.VMEM((CH,), _f32),
        pltpu.VMEM((RPT,), _f32),
        pltpu.VMEM_SHARED((NP,), _f32),
        pltpu.VMEM_SHARED((NP,), _f32),
        pltpu.VMEM_SHARED((NP,), _f32),
        pltpu.VMEM_SHARED((NP,), _f32),
        tuple(pltpu.SemaphoreType.DMA for _ in range(4)),
        pltpu.SemaphoreType.DMA,
    ),
)
def _sc_prep(v_ei, c_ei, p_idx, n_idx,
             pv_out, pc_out, nv_out, nc_out, cnt_out,
             ebuf, vall, call_, ones, zrows, h_pv, h_pc, h_nv, h_nc, gs, sh):
    cid = lax.axis_index("c")
    sid = lax.axis_index("s")
    pch = PCH0 + cid * (PCH1 - PCH0)
    rb = pl.multiple_of(cid * NS * PCH0 + sid * pch, 8)
    for q in range(CH // 16):
        ones[pl.ds(q * 16, 16)] = jnp.ones((16,), _f32)

    @pl.loop(0, RPT // 16)
    def _(q):
        zrows[pl.ds(q * 16, 16)] = jnp.zeros((16,), _f32)

    for h in (h_pv, h_pc, h_nv, h_nc):
        pltpu.sync_copy(zrows, h.at[pl.ds(sid * RPT, RPT)])
    plsc.subcore_barrier()

    def run(eidx, v_out, c_out, hv, hc):
        pltpu.sync_copy(eidx.at[pl.ds(rb, PCH_MAX)], ebuf)

        def fire(j, b):
            pltpu.async_copy(v_ei.at[ebuf.at[j]], vall.at[j], gs[b])
            pltpu.async_copy(c_ei.at[ebuf.at[j]], call_.at[j], gs[b])

        def drain(b):
            pltpu.make_async_copy(v_ei.at[ebuf.at[0]], vall.at[0],
                                  gs[b]).wait()
            pltpu.make_async_copy(c_ei.at[ebuf.at[0]], call_.at[0],
                                  gs[b]).wait()

        for b in range(3):
            fire(b, b)

        @pl.loop(0, pch // 4)
        def _(t):
            for b in range(4):
                j = 4 * t + b

                @pl.when(j + 3 < pch)
                def _():
                    fire(j + 3, (b + 3) % 4)

                drain(b)
                pltpu.async_copy(ones, hv.at[vall.at[j]], sh, add=True)
                pltpu.async_copy(ones, hc.at[call_.at[j]], sh, add=True)

        @pl.loop(0, 2 * pch)
        def _(j):
            pltpu.make_async_copy(ones, hv.at[vall.at[0]], sh).wait()

        pltpu.sync_copy(vall.at[pl.ds(0, PCH0)], v_out.at[pl.ds(rb, PCH0)])
        pltpu.sync_copy(call_.at[pl.ds(0, PCH0)], c_out.at[pl.ds(rb, PCH0)])

        @pl.when(pch > PCH0)
        def _():
            rb2 = pl.multiple_of(rb + PCH0, 8)
            pltpu.sync_copy(vall.at[pl.ds(PCH0, PCH_MAX - PCH0)],
                            v_out.at[pl.ds(rb2, PCH_MAX - PCH0)])
            pltpu.sync_copy(call_.at[pl.ds(PCH0, PCH_MAX - PCH0)],
                            c_out.at[pl.ds(rb2, PCH_MAX - PCH0)])

    run(p_idx, pv_out, pc_out, h_pv, h_pc)
    run(n_idx, nv_out, nc_out, h_nv, h_nc)
    plsc.subcore_barrier()
    for a, h in enumerate((h_pv, h_pc, h_nv, h_nc)):
        pltpu.sync_copy(h.at[pl.ds(sid * RPT, RPT)],
                        cnt_out.at[cid, a, pl.ds(sid * RPT, RPT)])


@functools.partial(
    pl.kernel,
    out_type=tuple(jax.ShapeDtypeStruct((NC, NP, D), _f32) for _ in range(4)),
    mesh=_mesh,
    scratch_types=(
        tuple(pltpu.VMEM((CH,), _i32) for _ in range(4)),
        tuple(pltpu.VMEM((CH,), _i32) for _ in range(4)),
        tuple(pltpu.VMEM((CH, D), _f32) for _ in range(2)),
        pltpu.VMEM((16, D), _f32),
        pltpu.VMEM_SHARED((NP, D), _f32),
        tuple(pltpu.SemaphoreType.DMA for _ in range(4)),
        tuple(pltpu.SemaphoreType.DMA for _ in range(2)),
        tuple(pltpu.SemaphoreType.DMA for _ in range(2)),
        pltpu.SemaphoreType.DMA,
    ),
)
def _sc_aggr(m_pv, m_nv, m_pc, m_nc, pv, pc, nv, nc,
             o_pvc, o_nvc, o_pcv, o_ncv,
             sbuf, dbuf, rows, zblk, acc, si, gs, ss, sz):
    cid = lax.axis_index("c")
    sid = lax.axis_index("s")
    nch = ACH0 + cid * (ACH1 - ACH0)
    eb = pl.multiple_of((cid * NS * ACH0 + sid * nch) * CH, 1024)

    @pl.loop(0, 16)
    def _(r):
        for q in range(D // 16):
            zblk[r, pl.ds(q * 16, 16)] = jnp.zeros((16,), _f32)

    jobs = ((m_pv, pv, pc, o_pvc),
            (m_nv, nv, nc, o_nvc),
            (m_pc, pc, pv, o_pcv),
            (m_nc, nc, nv, o_ncv))
    for tab, src, dst, out in jobs:
        @pl.loop(0, RPT // 16)
        def _(b):
            pltpu.async_copy(zblk, acc.at[pl.ds(sid * RPT + b * 16, 16)], sz)

        @pl.loop(0, RPT // 16)
        def _(b):
            pltpu.make_async_copy(zblk, acc.at[pl.ds(0, 16)], sz).wait()
        plsc.subcore_barrier()

        def startidx(j, q):
            pltpu.async_copy(src.at[pl.ds(eb + j * CH, CH)], sbuf[q], si[q])
            pltpu.async_copy(dst.at[pl.ds(eb + j * CH, CH)], dbuf[q], si[q])

        def startgather(j, b, q):
            pltpu.make_async_copy(src.at[pl.ds(0, CH)], sbuf[q],
                                  si[q]).wait()
            pltpu.make_async_copy(dst.at[pl.ds(0, CH)], dbuf[q],
                                  si[q]).wait()

            @pl.when(j >= 2)
            def _():
                pltpu.make_async_copy(rows[b], acc.at[dbuf[0]],
                                      ss[b]).wait()

            pltpu.async_copy(tab.at[sbuf[q]], rows[b], gs[b])

        def finish(b, q):
            pltpu.make_async_copy(tab.at[sbuf[q]], rows[b], gs[b]).wait()
            pltpu.async_copy(rows[b], acc.at[dbuf[q]], ss[b], add=True)

        startidx(0, 0)
        startidx(1, 1)
        startgather(0, 0, 0)

        @pl.loop(0, nch // 4)
        def _(t):
            for b in range(4):
                j = 4 * t + b

                @pl.when(j + 1 < nch)
                def _():
                    startgather(j + 1, (b + 1) % 2, (b + 1) % 4)

                finish(b % 2, b)

                @pl.when(j + 2 < nch)
                def _():
                    startidx(j + 2, (b + 2) % 4)

        for b in range(2):
            pltpu.make_async_copy(rows[b], acc.at[dbuf[0]], ss[b]).wait()
        plsc.subcore_barrier()
        pltpu.sync_copy(acc.at[pl.ds(sid * RPT, RPT)],
                        out.at[cid, pl.ds(sid * RPT, RPT)])
        plsc.subcore_barrier()


_BLK = 128
_G = NP // _BLK


def _dot(a, b):
    return jnp.dot(a, b, preferred_element_type=_f32)


def _msg_body(v_ref, c_ref, cnt_ref,
              pw1, pb1, pw2, pb2, nw1, nb1, nw2, nb2,
              qw1, qb1, qw2, qb2, rw1, rb1, rw2, rb2,
              o_pv, o_nv, o_pc, o_nc):
    cnt = cnt_ref[...]

    def scale(a):
        return lax.rsqrt(jnp.maximum(cnt[a] + cnt[4 + a], 1.0))

    def mlp(x, w1, b1, w2, b2):
        h = jnp.maximum(_dot(x, w1[...]) + b1[...], 0.0)
        return _dot(h, w2[...]) + b2[...]

    xv = v_ref[...]
    xc = c_ref[...]
    o_pv[...] = mlp(xv, pw1, pb1, pw2, pb2) * scale(0)[:, None]
    o_nv[...] = mlp(xv, nw1, nb1, nw2, nb2) * scale(2)[:, None]
    o_pc[...] = mlp(xc, qw1, qb1, qw2, qb2) * scale(1)[:, None]
    o_nc[...] = mlp(xc, rw1, rb1, rw2, rb2) * scale(3)[:, None]


def _tc_msg(vp, cp, cnt8, *ws):
    row = pl.BlockSpec((_BLK, D), lambda i: (i, 0))
    cnt = pl.BlockSpec((8, _BLK), lambda i: (0, i))
    w = pl.BlockSpec((D, D), lambda i: (0, 0))
    b = pl.BlockSpec((1, D), lambda i: (0, 0))
    return pl.pallas_call(
        _msg_body,
        grid=(_G,),
        in_specs=[row, row, cnt] + [w, b, w, b] * 4,
        out_specs=[row] * 4,
        out_shape=[jax.ShapeDtypeStruct((NP, D), _f32)] * 4,
    )(vp, cp, cnt8, *ws)


def _upd_body(c_ref, v_ref, a0, a1, a2, a3, cnt_ref,
              wc, bc, wv, bv, oc, ov):
    cnt = cnt_ref[...]

    def scale(a):
        return lax.rsqrt(jnp.maximum(cnt[a] + cnt[4 + a], 1.0))

    def agg(aref, a):
        x = aref[...]
        return (x[0] + x[1]) * scale(a)[:, None]

    wcm = wc[...]
    wvm = wv[...]
    oc[...] = (_dot(c_ref[...], wcm[0:D]) + _dot(agg(a0, 1), wcm[D:2 * D])
               + _dot(agg(a1, 3), wcm[2 * D:3 * D]) + bc[...])
    ov[...] = (_dot(v_ref[...], wvm[0:D]) + _dot(agg(a2, 0), wvm[D:2 * D])
               + _dot(agg(a3, 2), wvm[2 * D:3 * D]) + bv[...])


def _tc_upd(cp, vp, a_pvc, a_nvc, a_pcv, a_ncv, cnt8, wc, bc, wv, bv):
    row = pl.BlockSpec((_BLK, D), lambda i: (i, 0))
    aspec = pl.BlockSpec((NC, _BLK, D), lambda i: (0, i, 0))
    cnt = pl.BlockSpec((8, _BLK), lambda i: (0, i))
    w = pl.BlockSpec((3 * D, D), lambda i: (0, 0))
    b = pl.BlockSpec((1, D), lambda i: (0, 0))
    return pl.pallas_call(
        _upd_body,
        grid=(_G,),
        in_specs=[row, row, aspec, aspec, aspec, aspec, cnt, w, b, w, b],
        out_specs=[row, row],
        out_shape=[jax.ShapeDtypeStruct((NP, D), _f32)] * 2,
    )(cp, vp, a_pvc, a_nvc, a_pcv, a_ncv, cnt8, wc, bc, wv, bv)


def kernel(v_size, c_size, v_edge_index, c_edge_index, p_edge_index,
           n_edge_index, v_emb, c_emb,
           pv2c_W1, pv2c_b1, pv2c_W2, pv2c_b2,
           nv2c_W1, nv2c_b1, nv2c_W2, nv2c_b2,
           pc2v_W1, pc2v_b1, pc2v_W2, pc2v_b2,
           nc2v_W1, nc2v_b1, nc2v_W2, nc2v_b2,
           c_upd_W, c_upd_b, v_upd_W, v_upd_b):
    pad_t = V + (jnp.arange(PADT, dtype=_i32) % (NP - V))
    v_ei = jnp.concatenate([v_edge_index, pad_t])
    c_ei = jnp.concatenate([c_edge_index, pad_t])
    pad_e = E + (jnp.arange(ROWS_ALLOC * CH - EP, dtype=_i32) % PADT)
    pe = jnp.concatenate([p_edge_index, pad_e]).reshape(ROWS_ALLOC, CH)
    ne = jnp.concatenate([n_edge_index, pad_e]).reshape(ROWS_ALLOC, CH)

    pv, pc, nv, nc, cnt = _sc_prep(v_ei, c_ei, pe, ne)
    pv, pc, nv, nc = (x.reshape(EPAD) for x in (pv, pc, nv, nc))
    cnt8 = cnt.reshape(NC * 4, NP)

    zpad = jnp.zeros((NP - V, D), _f32)
    vp = jnp.concatenate([v_emb, zpad])
    cp = jnp.concatenate([c_emb, zpad])

    ws = (pv2c_W1, pv2c_b1.reshape(1, D), pv2c_W2, pv2c_b2.reshape(1, D),
          nv2c_W1, nv2c_b1.reshape(1, D), nv2c_W2, nv2c_b2.reshape(1, D),
          pc2v_W1, pc2v_b1.reshape(1, D), pc2v_W2, pc2v_b2.reshape(1, D),
          nc2v_W1, nc2v_b1.reshape(1, D), nc2v_W2, nc2v_b2.reshape(1, D))
    bc = c_upd_b.reshape(1, D)
    bv = v_upd_b.reshape(1, D)

    v_list = [vp]
    c_list = [cp]
    for _ in range(ITERS):
        m_pv, m_nv, m_pc, m_nc = _tc_msg(vp, cp, cnt8, *ws)
        a_pvc, a_nvc, a_pcv, a_ncv = _sc_aggr(m_pv, m_nv, m_pc, m_nc,
                                              pv, pc, nv, nc)
        cp, vp = _tc_upd(cp, vp, a_pvc, a_nvc, a_pcv, a_ncv, cnt8,
                         c_upd_W, bc, v_upd_W, bv)
        v_list.append(vp)
        c_list.append(cp)

    v_out = jnp.stack([x[:V] for x in v_list])
    c_out = jnp.stack([x[:V] for x in c_list])
    return (v_out, c_out)


# speedup vs baseline: 5.0864x; 1.0238x over previous
import functools

import jax
import jax.numpy as jnp
from jax import lax
from jax.experimental import pallas as pl
from jax.experimental.pallas import tpu as pltpu
from jax.experimental.pallas import tpu_sc as plsc

V = 10000
D = 128
E = 600000
EP = 300000
NP = 10240
NC = 2
NS = 16
NW = NC * NS
CH = 128
PCH0, PCH1 = 72, 80
ACH0, ACH1 = 76, 76
PCH_MAX = max(PCH0, PCH1)
NCH_MAX = max(ACH0, ACH1)
ROWS = NS * (PCH0 + PCH1)
EPAD = ROWS * CH
ROWS_ALLOC = ROWS
PADT = 2048
RPT = NP // NS
ITERS = 2
_f32 = jnp.float32
_i32 = jnp.int32

_mesh = plsc.VectorSubcoreMesh(core_axis_name="c", subcore_axis_name="s")


@functools.partial(
    pl.kernel,
    out_type=(
        jax.ShapeDtypeStruct((ROWS_ALLOC, CH), _i32),
        jax.ShapeDtypeStruct((ROWS_ALLOC, CH), _i32),
        jax.ShapeDtypeStruct((ROWS_ALLOC, CH), _i32),
        jax.ShapeDtypeStruct((ROWS_ALLOC, CH), _i32),
        jax.ShapeDtypeStruct((NC, 4, NP), _f32),
    ),
    mesh=_mesh,
    scratch_types=(
        pltpu.VMEM((PCH_MAX, CH), _i32),
        pltpu.VMEM((PCH_MAX, CH), _i32),
        pltpu.VMEM((PCH_MAX, CH), _i32),
        pltpu.VMEM((CH,), _f32),
        pltpu.VMEM((RPT,), _f32),
        pltpu.VMEM_SHARED((NP,), _f32),
        pltpu.VMEM_SHARED((NP,), _f32),
        pltpu.VMEM_SHARED((NP,), _f32),
        pltpu.VMEM_SHARED((NP,), _f32),
        tuple(pltpu.SemaphoreType.DMA for _ in range(4)),
        pltpu.SemaphoreType.DMA,
    ),
)
def _sc_prep(v_ei, c_ei, p_idx, n_idx,
             pv_out, pc_out, nv_out, nc_out, cnt_out,
             ebuf, vall, call_, ones, zrows, h_pv, h_pc, h_nv, h_nc, gs, sh):
    cid = lax.axis_index("c")
    sid = lax.axis_index("s")
    pch = PCH0 + cid * (PCH1 - PCH0)
    rb = pl.multiple_of(cid * NS * PCH0 + sid * pch, 8)
    for q in range(CH // 16):
        ones[pl.ds(q * 16, 16)] = jnp.ones((16,), _f32)

    @pl.loop(0, RPT // 16)
    def _(q):
        zrows[pl.ds(q * 16, 16)] = jnp.zeros((16,), _f32)

    for h in (h_pv, h_pc, h_nv, h_nc):
        pltpu.sync_copy(zrows, h.at[pl.ds(sid * RPT, RPT)])
    plsc.subcore_barrier()

    def run(eidx, v_out, c_out, hv, hc):
        pltpu.sync_copy(eidx.at[pl.ds(rb, PCH_MAX)], ebuf)

        def fire(j, b):
            pltpu.async_copy(v_ei.at[ebuf.at[j]], vall.at[j], gs[b])
            pltpu.async_copy(c_ei.at[ebuf.at[j]], call_.at[j], gs[b])

        def drain(b):
            pltpu.make_async_copy(v_ei.at[ebuf.at[0]], vall.at[0],
                                  gs[b]).wait()
            pltpu.make_async_copy(c_ei.at[ebuf.at[0]], call_.at[0],
                                  gs[b]).wait()

        for b in range(3):
            fire(b, b)

        @pl.loop(0, pch // 4)
        def _(t):
            for b in range(4):
                j = 4 * t + b

                @pl.when(j + 3 < pch)
                def _():
                    fire(j + 3, (b + 3) % 4)

                drain(b)
                pltpu.async_copy(ones, hv.at[vall.at[j]], sh, add=True)
                pltpu.async_copy(ones, hc.at[call_.at[j]], sh, add=True)

        @pl.loop(0, 2 * pch)
        def _(j):
            pltpu.make_async_copy(ones, hv.at[vall.at[0]], sh).wait()

        pltpu.sync_copy(vall.at[pl.ds(0, PCH0)], v_out.at[pl.ds(rb, PCH0)])
        pltpu.sync_copy(call_.at[pl.ds(0, PCH0)], c_out.at[pl.ds(rb, PCH0)])

        @pl.when(pch > PCH0)
        def _():
            rb2 = pl.multiple_of(rb + PCH0, 8)
            pltpu.sync_copy(vall.at[pl.ds(PCH0, PCH_MAX - PCH0)],
                            v_out.at[pl.ds(rb2, PCH_MAX - PCH0)])
            pltpu.sync_copy(call_.at[pl.ds(PCH0, PCH_MAX - PCH0)],
                            c_out.at[pl.ds(rb2, PCH_MAX - PCH0)])

    run(p_idx, pv_out, pc_out, h_pv, h_pc)
    run(n_idx, nv_out, nc_out, h_nv, h_nc)
    plsc.subcore_barrier()
    for a, h in enumerate((h_pv, h_pc, h_nv, h_nc)):
        pltpu.sync_copy(h.at[pl.ds(sid * RPT, RPT)],
                        cnt_out.at[cid, a, pl.ds(sid * RPT, RPT)])


@functools.partial(
    pl.kernel,
    out_type=tuple(jax.ShapeDtypeStruct((NC, NP, D), _f32) for _ in range(4)),
    mesh=_mesh,
    scratch_types=(
        tuple(pltpu.VMEM((CH,), _i32) for _ in range(4)),
        tuple(pltpu.VMEM((CH,), _i32) for _ in range(4)),
        tuple(pltpu.VMEM((CH, D), _f32) for _ in range(2)),
        pltpu.VMEM((16, D), _f32),
        pltpu.VMEM_SHARED((NP, D), _f32),
        tuple(pltpu.SemaphoreType.DMA for _ in range(4)),
        tuple(pltpu.SemaphoreType.DMA for _ in range(2)),
        tuple(pltpu.SemaphoreType.DMA for _ in range(2)),
        pltpu.SemaphoreType.DMA,
    ),
)
def _sc_aggr(m_pv, m_nv, m_pc, m_nc, pv, pc, nv, nc,
             o_pvc, o_nvc, o_pcv, o_ncv,
             sbuf, dbuf, rows, zblk, acc, si, gs, ss, sz):
    cid = lax.axis_index("c")
    sid = lax.axis_index("s")
    nch = ACH0 + cid * (ACH1 - ACH0)
    eb = pl.multiple_of((cid * NS * ACH0 + sid * nch) * CH, 512)

    @pl.loop(0, 16)
    def _(r):
        for q in range(D // 16):
            zblk[r, pl.ds(q * 16, 16)] = jnp.zeros((16,), _f32)

    jobs = ((m_pv, pv, pc, o_pvc),
            (m_nv, nv, nc, o_nvc),
            (m_pc, pc, pv, o_pcv),
            (m_nc, nc, nv, o_ncv))
    for tab, src, dst, out in jobs:
        @pl.loop(0, RPT // 16)
        def _(b):
            pltpu.async_copy(zblk, acc.at[pl.ds(sid * RPT + b * 16, 16)], sz)

        @pl.loop(0, RPT // 16)
        def _(b):
            pltpu.make_async_copy(zblk, acc.at[pl.ds(0, 16)], sz).wait()
        plsc.subcore_barrier()

        def startidx(j, q):
            pltpu.async_copy(src.at[pl.ds(eb + j * CH, CH)], sbuf[q], si[q])
            pltpu.async_copy(dst.at[pl.ds(eb + j * CH, CH)], dbuf[q], si[q])

        def startgather(j, b, q):
            pltpu.make_async_copy(src.at[pl.ds(0, CH)], sbuf[q],
                                  si[q]).wait()
            pltpu.make_async_copy(dst.at[pl.ds(0, CH)], dbuf[q],
                                  si[q]).wait()

            @pl.when(j >= 2)
            def _():
                pltpu.make_async_copy(rows[b], acc.at[dbuf[0]],
                                      ss[b]).wait()

            pltpu.async_copy(tab.at[sbuf[q]], rows[b], gs[b])

        def finish(b, q):
            pltpu.make_async_copy(tab.at[sbuf[q]], rows[b], gs[b]).wait()
            pltpu.async_copy(rows[b], acc.at[dbuf[q]], ss[b], add=True)

        startidx(0, 0)
        startidx(1, 1)
        startgather(0, 0, 0)

        @pl.loop(0, nch // 4)
        def _(t):
            for b in range(4):
                j = 4 * t + b

                @pl.when(j + 1 < nch)
                def _():
                    startgather(j + 1, (b + 1) % 2, (b + 1) % 4)

                finish(b % 2, b)

                @pl.when(j + 2 < nch)
                def _():
                    startidx(j + 2, (b + 2) % 4)

        for b in range(2):
            pltpu.make_async_copy(rows[b], acc.at[dbuf[0]], ss[b]).wait()
        plsc.subcore_barrier()
        pltpu.sync_copy(acc.at[pl.ds(sid * RPT, RPT)],
                        out.at[cid, pl.ds(sid * RPT, RPT)])
        plsc.subcore_barrier()


_BLK = 128
_G = NP // _BLK


def _dot(a, b):
    return jnp.dot(a, b, preferred_element_type=_f32)


def _msg_body(v_ref, c_ref, cnt_ref,
              pw1, pb1, pw2, pb2, nw1, nb1, nw2, nb2,
              qw1, qb1, qw2, qb2, rw1, rb1, rw2, rb2,
              o_pv, o_nv, o_pc, o_nc):
    cnt = cnt_ref[...]

    def scale(a):
        return lax.rsqrt(jnp.maximum(cnt[a] + cnt[4 + a], 1.0))

    def mlp(x, w1, b1, w2, b2):
        h = jnp.maximum(_dot(x, w1[...]) + b1[...], 0.0)
        return _dot(h, w2[...]) + b2[...]

    xv = v_ref[...]
    xc = c_ref[...]
    o_pv[...] = mlp(xv, pw1, pb1, pw2, pb2) * scale(0)[:, None]
    o_nv[...] = mlp(xv, nw1, nb1, nw2, nb2) * scale(2)[:, None]
    o_pc[...] = mlp(xc, qw1, qb1, qw2, qb2) * scale(1)[:, None]
    o_nc[...] = mlp(xc, rw1, rb1, rw2, rb2) * scale(3)[:, None]


def _tc_msg(vp, cp, cnt8, *ws):
    row = pl.BlockSpec((_BLK, D), lambda i: (i, 0))
    cnt = pl.BlockSpec((8, _BLK), lambda i: (0, i))
    w = pl.BlockSpec((D, D), lambda i: (0, 0))
    b = pl.BlockSpec((1, D), lambda i: (0, 0))
    return pl.pallas_call(
        _msg_body,
        grid=(_G,),
        in_specs=[row, row, cnt] + [w, b, w, b] * 4,
        out_specs=[row] * 4,
        out_shape=[jax.ShapeDtypeStruct((NP, D), _f32)] * 4,
    )(vp, cp, cnt8, *ws)


def _upd_body(c_ref, v_ref, a0, a1, a2, a3, cnt_ref,
              wc, bc, wv, bv, oc, ov):
    cnt = cnt_ref[...]

    def scale(a):
        return lax.rsqrt(jnp.maximum(cnt[a] + cnt[4 + a], 1.0))

    def agg(aref, a):
        x = aref[...]
        return (x[0] + x[1]) * scale(a)[:, None]

    wcm = wc[...]
    wvm = wv[...]
    oc[...] = (_dot(c_ref[...], wcm[0:D]) + _dot(agg(a0, 1), wcm[D:2 * D])
               + _dot(agg(a1, 3), wcm[2 * D:3 * D]) + bc[...])
    ov[...] = (_dot(v_ref[...], wvm[0:D]) + _dot(agg(a2, 0), wvm[D:2 * D])
               + _dot(agg(a3, 2), wvm[2 * D:3 * D]) + bv[...])


def _tc_upd(cp, vp, a_pvc, a_nvc, a_pcv, a_ncv, cnt8, wc, bc, wv, bv):
    row = pl.BlockSpec((_BLK, D), lambda i: (i, 0))
    aspec = pl.BlockSpec((NC, _BLK, D), lambda i: (0, i, 0))
    cnt = pl.BlockSpec((8, _BLK), lambda i: (0, i))
    w = pl.BlockSpec((3 * D, D), lambda i: (0, 0))
    b = pl.BlockSpec((1, D), lambda i: (0, 0))
    return pl.pallas_call(
        _upd_body,
        grid=(_G,),
        in_specs=[row, row, aspec, aspec, aspec, aspec, cnt, w, b, w, b],
        out_specs=[row, row],
        out_shape=[jax.ShapeDtypeStruct((NP, D), _f32)] * 2,
    )(cp, vp, a_pvc, a_nvc, a_pcv, a_ncv, cnt8, wc, bc, wv, bv)


def kernel(v_size, c_size, v_edge_index, c_edge_index, p_edge_index,
           n_edge_index, v_emb, c_emb,
           pv2c_W1, pv2c_b1, pv2c_W2, pv2c_b2,
           nv2c_W1, nv2c_b1, nv2c_W2, nv2c_b2,
           pc2v_W1, pc2v_b1, pc2v_W2, pc2v_b2,
           nc2v_W1, nc2v_b1, nc2v_W2, nc2v_b2,
           c_upd_W, c_upd_b, v_upd_W, v_upd_b):
    pad_t = V + (jnp.arange(PADT, dtype=_i32) % (NP - V))
    v_ei = jnp.concatenate([v_edge_index, pad_t])
    c_ei = jnp.concatenate([c_edge_index, pad_t])
    pad_e = E + (jnp.arange(ROWS_ALLOC * CH - EP, dtype=_i32) % PADT)
    pe = jnp.concatenate([p_edge_index, pad_e]).reshape(ROWS_ALLOC, CH)
    ne = jnp.concatenate([n_edge_index, pad_e]).reshape(ROWS_ALLOC, CH)

    pv, pc, nv, nc, cnt = _sc_prep(v_ei, c_ei, pe, ne)
    pv, pc, nv, nc = (x.reshape(EPAD) for x in (pv, pc, nv, nc))
    cnt8 = cnt.reshape(NC * 4, NP)

    zpad = jnp.zeros((NP - V, D), _f32)
    vp = jnp.concatenate([v_emb, zpad])
    cp = jnp.concatenate([c_emb, zpad])

    ws = (pv2c_W1, pv2c_b1.reshape(1, D), pv2c_W2, pv2c_b2.reshape(1, D),
          nv2c_W1, nv2c_b1.reshape(1, D), nv2c_W2, nv2c_b2.reshape(1, D),
          pc2v_W1, pc2v_b1.reshape(1, D), pc2v_W2, pc2v_b2.reshape(1, D),
          nc2v_W1, nc2v_b1.reshape(1, D), nc2v_W2, nc2v_b2.reshape(1, D))
    bc = c_upd_b.reshape(1, D)
    bv = v_upd_b.reshape(1, D)

    v_list = [vp]
    c_list = [cp]
    for _ in range(ITERS):
        m_pv, m_nv, m_pc, m_nc = _tc_msg(vp, cp, cnt8, *ws)
        a_pvc, a_nvc, a_pcv, a_ncv = _sc_aggr(m_pv, m_nv, m_pc, m_nc,
                                              pv, pc, nv, nc)
        cp, vp = _tc_upd(cp, vp, a_pvc, a_nvc, a_pcv, a_ncv, cnt8,
                         c_upd_W, bc, v_upd_W, bv)
        v_list.append(vp)
        c_list.append(cp)

    v_out = jnp.stack([x[:V] for x in v_list])
    c_out = jnp.stack([x[:V] for x in c_list])
    return (v_out, c_out)
